# Initial kernel scaffold; baseline (speedup 1.0000x reference)
import jax
import jax.numpy as jnp
from jax.experimental import pallas as pl


def kernel(input, hidden, edge_index, edge_weight, W, b):
    raise NotImplementedError("write your pallas kernel here")


# speedup vs baseline: 3.5254x; 3.5254x over previous
import functools

import jax
import jax.numpy as jnp
from jax import lax
from jax.experimental import pallas as pl
from jax.experimental.pallas import tpu as pltpu
from jax.experimental.pallas import tpu_sc as plsc
from jax._src.pallas.mosaic import helpers as plm_helpers

N = 10000
E = 320000
D = 128
K = 10
NC = 2
NS = 16
L = 16
CHUNK = 128
EPT = 20096
NCHUNK = EPT // CHUNK
E_PAD = EPT * NS
N_PAD = 10240
ROWS_PT = N_PAD // NS
NF = (K + 1) * NC
TROWS = NF * N_PAD


def _sc_diffusion_body(t0_hbm, src_hbm, dst_hbm, w_hbm, zero_hbm, tstk_hbm,
                       acc_sh, src_v, dst_v, w_v, idx_v, rows_v, sem, sem_e):
    c = lax.axis_index("c")
    s = lax.axis_index("s")
    rbase = pl.multiple_of(s * ROWS_PT, 128)
    ebase = pl.multiple_of(s * EPT, 128)

    plm_helpers.sync_copy(
        t0_hbm.at[c, pl.ds(rbase, ROWS_PT)],
        tstk_hbm.at[pl.ds(pl.multiple_of(c * N_PAD + rbase, 128), ROWS_PT)])
    plsc.subcore_barrier()

    def chunk_body(j, base):
        e0 = ebase + j * CHUNK
        cp_s = pltpu.async_copy(src_hbm.at[pl.ds(e0, CHUNK)], src_v.at[0],
                                sem_e)
        cp_d = pltpu.async_copy(dst_hbm.at[pl.ds(e0, CHUNK)], dst_v.at[0],
                                sem_e)
        cp_w = pltpu.async_copy(w_hbm.at[pl.ds(e0, CHUNK)], w_v.at[0],
                                sem_e)
        cp_s.wait()
        cp_d.wait()
        cp_w.wait()

        for q in range(CHUNK // L):
            sl = pl.ds(q * L, L)
            idx_v[0, sl] = src_v[0, sl] + base

        pltpu.async_copy(tstk_hbm.at[idx_v.at[0]], rows_v, sem).wait()

        def group_body(g, carry2):
            wvec = w_v[0, pl.ds(g * L, L)]
            for l in range(L):
                e = g * L + l
                wsc = wvec[l]
                for q in range(D // L):
                    sl = pl.ds(q * L, L)
                    rows_v[e, sl] = rows_v[e, sl] * wsc
            return carry2
        lax.fori_loop(0, CHUNK // L, group_body, 0)

        pltpu.async_copy(rows_v, acc_sh.at[dst_v.at[0]], sem,
                         add=True).wait()
        return base

    def step_body(k, carry):
        plm_helpers.sync_copy(zero_hbm, acc_sh.at[pl.ds(rbase, ROWS_PT)])
        plsc.subcore_barrier()
        gbase = ((k - 1) * NC + c) * N_PAD
        lax.fori_loop(0, NCHUNK, chunk_body, gbase)
        plsc.subcore_barrier()
        dbase = pl.multiple_of((k * NC + c) * N_PAD + rbase, 128)
        plm_helpers.sync_copy(
            acc_sh.at[pl.ds(rbase, ROWS_PT)],
            tstk_hbm.at[pl.ds(dbase, ROWS_PT)])
        plsc.subcore_barrier()
        return carry

    lax.fori_loop(1, K + 1, step_body, 0)


@functools.cache
def _sc_diffusion_kernel():
    return pl.kernel(
        _sc_diffusion_body,
        out_type=jax.ShapeDtypeStruct((TROWS, D), jnp.float32),
        mesh=plsc.VectorSubcoreMesh(core_axis_name="c", subcore_axis_name="s",
                                    num_cores=NC, num_subcores=NS),
        scratch_types=[
            pltpu.VMEM_SHARED((N_PAD, D), jnp.float32),
            pltpu.VMEM((1, CHUNK), jnp.int32),
            pltpu.VMEM((1, CHUNK), jnp.int32),
            pltpu.VMEM((1, CHUNK), jnp.float32),
            pltpu.VMEM((1, CHUNK), jnp.int32),
            pltpu.VMEM((CHUNK, D), jnp.float32),
            pltpu.SemaphoreType.DMA,
            pltpu.SemaphoreType.DMA,
        ],
        compiler_params=pltpu.CompilerParams(use_tc_tiling_on_sc=False),
    )


def _sc_diffusion(t0, src, dst, wgt, zero):
    return _sc_diffusion_kernel()(t0, src, dst, wgt, zero)


BN = 256


def _concat_feats(t_ref):
    t = t_ref[...]
    return jnp.concatenate([t[i] for i in range(NF)], axis=1)


def _mm1_body(t_ref, w_ref, b_ref, h_ref, g_ref, rh_ref):
    h_cat = _concat_feats(t_ref)
    g = jax.nn.sigmoid(
        jnp.dot(h_cat, w_ref[...], preferred_element_type=jnp.float32)
        + b_ref[...])
    g_ref[...] = g
    rh_ref[...] = g * h_ref[...]


def _mm2_body(t_ref, w_ref, b_ref, g_ref, h_ref, o_ref):
    h_cat = _concat_feats(t_ref)
    cand = jax.nn.sigmoid(
        jnp.dot(h_cat, w_ref[...], preferred_element_type=jnp.float32)
        + b_ref[...])
    g = g_ref[...]
    o_ref[...] = g * h_ref[...] + (1.0 - g) * cand


_t_spec = pl.BlockSpec((NF, BN, D), lambda i: (0, i, 0))
_row_spec = pl.BlockSpec((BN, D), lambda i: (i, 0))
_b_spec = pl.BlockSpec((1, D), lambda i: (0, 0))
_w_spec = pl.BlockSpec(((K + 1) * 2 * D, D), lambda i: (0, 0))
_GRID = (N_PAD // BN,)


def _mm1(tstk, W, b2, hidden):
    return pl.pallas_call(
        _mm1_body,
        grid=_GRID,
        in_specs=[_t_spec, _w_spec, _b_spec, _row_spec],
        out_specs=[_row_spec, _row_spec],
        out_shape=[jax.ShapeDtypeStruct((N, D), jnp.float32),
                   jax.ShapeDtypeStruct((N_PAD, D), jnp.float32)],
    )(tstk, W, b2, hidden)


def _mm2(tstk, W, b2, g, hidden):
    return pl.pallas_call(
        _mm2_body,
        grid=_GRID,
        in_specs=[_t_spec, _w_spec, _b_spec, _row_spec, _row_spec],
        out_specs=_row_spec,
        out_shape=jax.ShapeDtypeStruct((N, D), jnp.float32),
    )(tstk, W, b2, g, hidden)


def kernel(input, hidden, edge_index, edge_weight, W, b):
    b2 = b.reshape(1, D)

    pad = E_PAD - E
    src = jnp.pad(edge_index[0], (0, pad))
    dst = jnp.pad(edge_index[1], (0, pad))
    wgt = jnp.pad(edge_weight, (0, pad))
    zero = jnp.zeros((ROWS_PT, D), jnp.float32)

    x_pad = jnp.pad(input, ((0, N_PAD - N), (0, 0)))
    h_pad = jnp.pad(hidden, ((0, N_PAD - N), (0, 0)))

    t0_1 = jnp.stack([x_pad, h_pad])
    tstk1 = _sc_diffusion(t0_1, src, dst, wgt, zero)
    g, rh = _mm1(tstk1.reshape(NF, N_PAD, D), W, b2, hidden)

    t0_2 = jnp.stack([x_pad, rh])
    tstk2 = _sc_diffusion(t0_2, src, dst, wgt, zero)
    out = _mm2(tstk2.reshape(NF, N_PAD, D), W, b2, g, hidden)
    return (out, out)


# speedup vs baseline: 3.6714x; 1.0414x over previous
import functools

import jax
import jax.numpy as jnp
from jax import lax
from jax.experimental import pallas as pl
from jax.experimental.pallas import tpu as pltpu
from jax.experimental.pallas import tpu_sc as plsc
from jax._src.pallas.mosaic import helpers as plm_helpers

N = 10000
E = 320000
D = 128
K = 10
NC = 2
NS = 16
L = 16
CHUNK = 128
EPT = 20224
NCHUNK = EPT // CHUNK
NPAIR = NCHUNK // 2
E_PAD = EPT * NS
N_PAD = 10240
ROWS_PT = N_PAD // NS
NF = (K + 1) * NC
TROWS = NF * N_PAD


def _sc_diffusion_body(t0_hbm, src_hbm, dst_hbm, w_hbm, zero_hbm, tstk_hbm,
                       acc_sh, src_v, dst_v, w_v, idx_v, rows_a, rows_b,
                       sem_e, sem_g1, sem_g2, sem_s1, sem_s2):
    c = lax.axis_index("c")
    s = lax.axis_index("s")
    rbase = pl.multiple_of(s * ROWS_PT, 128)
    ebase = pl.multiple_of(s * EPT, 128)

    plm_helpers.sync_copy(
        t0_hbm.at[c, pl.ds(rbase, ROWS_PT)],
        tstk_hbm.at[pl.ds(pl.multiple_of(c * N_PAD + rbase, 128), ROWS_PT)])
    plsc.subcore_barrier()

    def scale_rows(rows_ref, h):
        def group_body(g, carry2):
            wvec = w_v[h, pl.ds(g * L, L)]
            for l in range(L):
                e = g * L + l
                wsc = wvec[l]
                for q in range(D // L):
                    sl = pl.ds(q * L, L)
                    rows_ref[e, sl] = rows_ref[e, sl] * wsc
            return carry2
        lax.fori_loop(0, CHUNK // L, group_body, 0)

    def pair_body(i, base):
        e0 = ebase + i * (2 * CHUNK)
        cps = [
            pltpu.async_copy(src_hbm.at[pl.ds(e0, CHUNK)], src_v.at[0],
                             sem_e),
            pltpu.async_copy(src_hbm.at[pl.ds(e0 + CHUNK, CHUNK)],
                             src_v.at[1], sem_e),
            pltpu.async_copy(dst_hbm.at[pl.ds(e0, CHUNK)], dst_v.at[0],
                             sem_e),
            pltpu.async_copy(dst_hbm.at[pl.ds(e0 + CHUNK, CHUNK)],
                             dst_v.at[1], sem_e),
            pltpu.async_copy(w_hbm.at[pl.ds(e0, CHUNK)], w_v.at[0], sem_e),
            pltpu.async_copy(w_hbm.at[pl.ds(e0 + CHUNK, CHUNK)],
                             w_v.at[1], sem_e),
        ]
        for cp in cps:
            cp.wait()

        for h in range(2):
            for q in range(CHUNK // L):
                sl = pl.ds(q * L, L)
                idx_v[h, sl] = src_v[h, sl] + base

        ga = pltpu.async_copy(tstk_hbm.at[idx_v.at[0]], rows_a, sem_g1)
        gb = pltpu.async_copy(tstk_hbm.at[idx_v.at[1]], rows_b, sem_g2)
        ga.wait()
        scale_rows(rows_a, 0)
        sa = pltpu.async_copy(rows_a, acc_sh.at[dst_v.at[0]], sem_s1,
                              add=True)
        gb.wait()
        scale_rows(rows_b, 1)
        sb = pltpu.async_copy(rows_b, acc_sh.at[dst_v.at[1]], sem_s2,
                              add=True)
        sa.wait()
        sb.wait()
        return base

    def step_body(k, carry):
        plm_helpers.sync_copy(zero_hbm, acc_sh.at[pl.ds(rbase, ROWS_PT)])
        plsc.subcore_barrier()
        gbase = ((k - 1) * NC + c) * N_PAD
        lax.fori_loop(0, NPAIR, pair_body, gbase)
        plsc.subcore_barrier()
        dbase = pl.multiple_of((k * NC + c) * N_PAD + rbase, 128)
        plm_helpers.sync_copy(
            acc_sh.at[pl.ds(rbase, ROWS_PT)],
            tstk_hbm.at[pl.ds(dbase, ROWS_PT)])
        plsc.subcore_barrier()
        return carry

    lax.fori_loop(1, K + 1, step_body, 0)


@functools.cache
def _sc_diffusion_kernel():
    return pl.kernel(
        _sc_diffusion_body,
        out_type=jax.ShapeDtypeStruct((TROWS, D), jnp.float32),
        mesh=plsc.VectorSubcoreMesh(core_axis_name="c", subcore_axis_name="s",
                                    num_cores=NC, num_subcores=NS),
        scratch_types=[
            pltpu.VMEM_SHARED((N_PAD, D), jnp.float32),
            pltpu.VMEM((2, CHUNK), jnp.int32),
            pltpu.VMEM((2, CHUNK), jnp.int32),
            pltpu.VMEM((2, CHUNK), jnp.float32),
            pltpu.VMEM((2, CHUNK), jnp.int32),
            pltpu.VMEM((CHUNK, D), jnp.float32),
            pltpu.VMEM((CHUNK, D), jnp.float32),
            pltpu.SemaphoreType.DMA,
            pltpu.SemaphoreType.DMA,
            pltpu.SemaphoreType.DMA,
            pltpu.SemaphoreType.DMA,
            pltpu.SemaphoreType.DMA,
        ],
        compiler_params=pltpu.CompilerParams(use_tc_tiling_on_sc=False),
    )


def _sc_diffusion(t0, src, dst, wgt, zero):
    return _sc_diffusion_kernel()(t0, src, dst, wgt, zero)


BN = 256


def _concat_feats(t_ref):
    t = t_ref[...]
    return jnp.concatenate([t[i] for i in range(NF)], axis=1)


def _mm1_body(t_ref, w_ref, b_ref, h_ref, g_ref, rh_ref):
    h_cat = _concat_feats(t_ref)
    g = jax.nn.sigmoid(
        jnp.dot(h_cat, w_ref[...], preferred_element_type=jnp.float32)
        + b_ref[...])
    g_ref[...] = g
    rh_ref[...] = g * h_ref[...]


def _mm2_body(t_ref, w_ref, b_ref, g_ref, h_ref, o_ref):
    h_cat = _concat_feats(t_ref)
    cand = jax.nn.sigmoid(
        jnp.dot(h_cat, w_ref[...], preferred_element_type=jnp.float32)
        + b_ref[...])
    g = g_ref[...]
    o_ref[...] = g * h_ref[...] + (1.0 - g) * cand


_t_spec = pl.BlockSpec((NF, BN, D), lambda i: (0, i, 0))
_row_spec = pl.BlockSpec((BN, D), lambda i: (i, 0))
_b_spec = pl.BlockSpec((1, D), lambda i: (0, 0))
_w_spec = pl.BlockSpec(((K + 1) * 2 * D, D), lambda i: (0, 0))
_GRID = (N_PAD // BN,)


def _mm1(tstk, W, b2, hidden):
    return pl.pallas_call(
        _mm1_body,
        grid=_GRID,
        in_specs=[_t_spec, _w_spec, _b_spec, _row_spec],
        out_specs=[_row_spec, _row_spec],
        out_shape=[jax.ShapeDtypeStruct((N, D), jnp.float32),
                   jax.ShapeDtypeStruct((N_PAD, D), jnp.float32)],
    )(tstk, W, b2, hidden)


def _mm2(tstk, W, b2, g, hidden):
    return pl.pallas_call(
        _mm2_body,
        grid=_GRID,
        in_specs=[_t_spec, _w_spec, _b_spec, _row_spec, _row_spec],
        out_specs=_row_spec,
        out_shape=jax.ShapeDtypeStruct((N, D), jnp.float32),
    )(tstk, W, b2, g, hidden)


def kernel(input, hidden, edge_index, edge_weight, W, b):
    b2 = b.reshape(1, D)

    pad = E_PAD - E
    src = jnp.pad(edge_index[0], (0, pad))
    dst = jnp.pad(edge_index[1], (0, pad))
    wgt = jnp.pad(edge_weight, (0, pad))
    zero = jnp.zeros((ROWS_PT, D), jnp.float32)

    x_pad = jnp.pad(input, ((0, N_PAD - N), (0, 0)))
    h_pad = jnp.pad(hidden, ((0, N_PAD - N), (0, 0)))

    t0_1 = jnp.stack([x_pad, h_pad])
    tstk1 = _sc_diffusion(t0_1, src, dst, wgt, zero)
    g, rh = _mm1(tstk1.reshape(NF, N_PAD, D), W, b2, hidden)

    t0_2 = jnp.stack([x_pad, rh])
    tstk2 = _sc_diffusion(t0_2, src, dst, wgt, zero)
    out = _mm2(tstk2.reshape(NF, N_PAD, D), W, b2, g, hidden)
    return (out, out)
